# baseline (speedup 1.0000x reference)
import functools

import jax
import jax.numpy as jnp
from jax import lax
from jax.experimental import pallas as pl
from jax.experimental.pallas import tpu as pltpu
from jax.experimental.pallas import tpu_sc as plsc

N = 10000
E = 320000
D = 128
DE = 16
DEP = 128

NC = 2
NS = 16
NW = NC * NS
K = 80
EP = 327680
EPW = EP // NW
NCHUNK = EPW // K
NP = 10240
PAD_DST = N + 8
RPS = NP // NS

BR = 1000
G = N // BR


def _sc_mesh():
    return plsc.VectorSubcoreMesh(core_axis_name="c", subcore_axis_name="s")


def _gather_segsum(h, src3, dst3, zeros):
    @functools.partial(
        pl.kernel,
        mesh=_sc_mesh(),
        out_type=jax.ShapeDtypeStruct((NC, NP, D), jnp.float32),
        scratch_types=[
            pltpu.VMEM((NCHUNK, K), jnp.int32),
            pltpu.VMEM((NCHUNK, K), jnp.int32),
            pltpu.VMEM((K, D), jnp.float32),
            pltpu.VMEM_SHARED((NP, D), jnp.float32),
            pltpu.SemaphoreType.DMA,
        ],
    )
    def seg(h_hbm, src_hbm, dst_hbm, z_hbm, out_hbm,
            sidx, didx, rows, acc, sem):
        c = lax.axis_index("c")
        s = lax.axis_index("s")
        wid = s * NC + c
        pltpu.sync_copy(src_hbm.at[wid], sidx)
        pltpu.sync_copy(dst_hbm.at[wid], didx)
        pltpu.sync_copy(z_hbm, acc.at[pl.ds(s * RPS, RPS)])
        plsc.subcore_barrier()

        def body(j, carry):
            pltpu.async_copy(h_hbm.at[sidx.at[j]], rows, sem).wait()
            pltpu.sync_copy(rows, acc.at[didx.at[j]], add=True)
            return carry

        lax.fori_loop(0, NCHUNK, body, 0)
        plsc.subcore_barrier()
        pltpu.sync_copy(acc.at[pl.ds(s * RPS, RPS)],
                        out_hbm.at[c, pl.ds(s * RPS, RPS)])

    return seg(h, src3, dst3, zeros)


def _p_spec():
    return pl.BlockSpec((NC, BR, D), lambda i: (0, i, 0))


def _ea_spec():
    return pl.BlockSpec((NC, BR, DEP), lambda i: (0, i, 0))


def _row_spec(d=D):
    return pl.BlockSpec((BR, d), lambda i: (i, 0))


def _full_spec(a, b):
    return pl.BlockSpec((a, b), lambda i: (0, 0))


def _smem_spec(n):
    return pl.BlockSpec(memory_space=pltpu.SMEM)


def _tc_matmul(x, w):
    def body(x_ref, w_ref, o_ref):
        o_ref[...] = jnp.dot(x_ref[...], w_ref[...],
                             preferred_element_type=jnp.float32)

    return pl.pallas_call(
        body,
        grid=(G,),
        in_specs=[_row_spec(), _full_spec(D, D)],
        out_specs=_row_spec(),
        out_shape=jax.ShapeDtypeStruct((N, D), jnp.float32),
    )(x, w)


def _tc_step(p, eap, we, wn, terms, skw):
    nt = len(terms)

    def body(*refs):
        p_ref, ea_ref, we_ref, wn_ref = refs[:4]
        t_refs = refs[4:4 + nt]
        skw_ref = refs[4 + nt]
        t_ref, h_ref = refs[5 + nt:]
        ea = ea_ref[0] + ea_ref[1]
        agg = (p_ref[0] + p_ref[1]
               + jnp.dot(ea, we_ref[...], preferred_element_type=jnp.float32))
        xk = jnp.maximum(agg, 0.0)
        if nt:
            xkw = skw_ref[0] * t_refs[0][...]
            for j in range(1, nt):
                xkw = xkw + skw_ref[j] * t_refs[j][...]
            xkw = xkw + skw_ref[nt] * xk
        else:
            xkw = xk
        t_ref[...] = xkw
        h_ref[...] = jnp.dot(xkw, wn_ref[...],
                             preferred_element_type=jnp.float32)

    return pl.pallas_call(
        body,
        grid=(G,),
        in_specs=[_p_spec(), _ea_spec(), _full_spec(DEP, D), _full_spec(D, D)]
                 + [_row_spec() for _ in range(nt)] + [_smem_spec(nt + 1)],
        out_specs=[_row_spec(), _row_spec()],
        out_shape=[jax.ShapeDtypeStruct((N, D), jnp.float32),
                   jax.ShapeDtypeStruct((N, D), jnp.float32)],
    )(p, eap, we, wn, *terms, skw)


def _tc_last(p, eap, we):
    def body(p_ref, ea_ref, we_ref, o_ref):
        ea = ea_ref[0] + ea_ref[1]
        agg = (p_ref[0] + p_ref[1]
               + jnp.dot(ea, we_ref[...], preferred_element_type=jnp.float32))
        o_ref[...] = jnp.maximum(agg, 0.0)

    return pl.pallas_call(
        body,
        grid=(G,),
        in_specs=[_p_spec(), _ea_spec(), _full_spec(DEP, D)],
        out_specs=_row_spec(),
        out_shape=jax.ShapeDtypeStruct((N, D), jnp.float32),
    )(p, eap, we)


def kernel(x, edge_index, edge_attr, params):
    L = params['layers']
    w = params['skip']

    npad = EP - E
    pad_dst = N + (jnp.arange(npad, dtype=jnp.int32) % (NP - N))
    src3 = jnp.concatenate(
        [edge_index[0], jnp.zeros((npad,), jnp.int32)]).reshape(NW, NCHUNK, K)
    dst3 = jnp.concatenate(
        [edge_index[1], pad_dst]).reshape(NW, NCHUNK, K)
    eidx3 = jnp.concatenate(
        [jnp.arange(E, dtype=jnp.int32), jnp.zeros((npad,), jnp.int32)]
    ).reshape(NW, NCHUNK, K)
    ea2 = jnp.concatenate(
        [edge_attr,
         jnp.ones((E, 1), jnp.float32),
         jnp.zeros((E, DEP - DE - 1), jnp.float32)], axis=1)
    z128 = jnp.zeros((RPS, D), jnp.float32)

    def packed_we(l):
        p = L[l]
        return (jnp.zeros((DEP, D), jnp.float32)
                .at[:DE].set(p['We'])
                .at[DE].set(p['bn'] + p['be']))

    eap = _gather_segsum(ea2, eidx3, dst3, z128)

    agg_layers = [0, 1, 2, 3, 3, 4, 5]
    nxt_layers = [1, 2, 3, 3, 4, 5, 7]
    skips = [
        [],
        [w['w2_1'], w['w2_2']],
        [w['w3_1'], w['w3_2'], w['w3_3']],
        [w['w4_1'], w['w4_2'], w['w4_3'], w['w4_4']],
        [w['w5_1'], w['w5_2'], w['w5_3'], w['w5_4'], w['w5_5']],
        [w['w6_1'], w['w6_2'], w['w6_3'], w['w6_4'], w['w6_5'], w['w6_6']],
        [w['w7_1'], w['w7_2'], w['w7_3'], w['w7_4'], w['w7_5'], w['w7_6'],
         w['w7_7']],
    ]

    h = _tc_matmul(x, L[0]['Wn'])
    terms = []
    for k in range(7):
        p = _gather_segsum(h, src3, dst3, z128)
        skw = jnp.stack(skips[k]) if skips[k] else jnp.ones((1,), jnp.float32)
        xkw, h = _tc_step(p, eap, packed_we(agg_layers[k]),
                          L[nxt_layers[k]]['Wn'], terms, skw)
        terms.append(xkw)
    p = _gather_segsum(h, src3, dst3, z128)
    return _tc_last(p, eap, packed_we(7))


# speedup vs baseline: 2.3419x; 2.3419x over previous
import functools

import jax
import jax.numpy as jnp
from jax import lax
from jax.experimental import pallas as pl
from jax.experimental.pallas import tpu as pltpu
from jax.experimental.pallas import tpu_sc as plsc

N = 10000
E = 320000
D = 128
DE = 16
DEP = 128

NC = 2
NS = 16
NW = NC * NS
K = 80
EP = 327680
EPW = EP // NW
NCHUNK = EPW // K
NP = 10240
PAD_DST = N + 8
RPS = NP // NS

BR = 1000
G = N // BR


def _sc_mesh():
    return plsc.VectorSubcoreMesh(core_axis_name="c", subcore_axis_name="s")


def _gather_segsum(h, src3, dst3, zeros):
    @functools.partial(
        pl.kernel,
        mesh=_sc_mesh(),
        out_type=jax.ShapeDtypeStruct((NC, NP, D), jnp.float32),
        scratch_types=[
            pltpu.VMEM((NCHUNK, K), jnp.int32),
            pltpu.VMEM((NCHUNK, K), jnp.int32),
            pltpu.VMEM((K, D), jnp.float32),
            pltpu.VMEM_SHARED((NP, D), jnp.float32),
            pltpu.SemaphoreType.DMA,
        ],
    )
    def seg(h_hbm, src_hbm, dst_hbm, z_hbm, out_hbm,
            sidx, didx, rows, acc, sem):
        c = lax.axis_index("c")
        s = lax.axis_index("s")
        wid = s * NC + c
        pltpu.sync_copy(src_hbm.at[wid], sidx)
        pltpu.sync_copy(dst_hbm.at[wid], didx)
        pltpu.sync_copy(z_hbm, acc.at[pl.ds(s * RPS, RPS)])
        plsc.subcore_barrier()

        def body(j, carry):
            pltpu.async_copy(h_hbm.at[sidx.at[j]], rows, sem).wait()
            pltpu.sync_copy(rows, acc.at[didx.at[j]], add=True)
            return carry

        lax.fori_loop(0, NCHUNK, body, 0)
        plsc.subcore_barrier()
        pltpu.sync_copy(acc.at[pl.ds(s * RPS, RPS)],
                        out_hbm.at[c, pl.ds(s * RPS, RPS)])

    return seg(h, src3, dst3, zeros)


def _p_spec():
    return pl.BlockSpec((NC, BR, D), lambda i: (0, i, 0))


def _ea_spec():
    return pl.BlockSpec((NC, BR, DEP), lambda i: (0, i, 0))


def _row_spec(d=D):
    return pl.BlockSpec((BR, d), lambda i: (i, 0))


def _full_spec(a, b):
    return pl.BlockSpec((a, b), lambda i: (0, 0))


def _smem_spec(n):
    return pl.BlockSpec(memory_space=pltpu.SMEM)


def _tc_matmul(x, w):
    def body(x_ref, w_ref, o_ref):
        o_ref[...] = jnp.dot(x_ref[...], w_ref[...],
                             preferred_element_type=jnp.float32)

    return pl.pallas_call(
        body,
        grid=(G,),
        in_specs=[_row_spec(), _full_spec(D, D)],
        out_specs=_row_spec(),
        out_shape=jax.ShapeDtypeStruct((N, D), jnp.float32),
    )(x, w)


def _tc_step(p, eap, we, wn, terms, skw):
    nt = len(terms)

    def body(*refs):
        p_ref, ea_ref, we_ref, wn_ref = refs[:4]
        t_refs = refs[4:4 + nt]
        skw_ref = refs[4 + nt]
        t_ref, h_ref = refs[5 + nt:]
        ea = ea_ref[0] + ea_ref[1]
        agg = (p_ref[0] + p_ref[1]
               + jnp.dot(ea, we_ref[...], preferred_element_type=jnp.float32))
        xk = jnp.maximum(agg, 0.0)
        if nt:
            xkw = skw_ref[0] * t_refs[0][...]
            for j in range(1, nt):
                xkw = xkw + skw_ref[j] * t_refs[j][...]
            xkw = xkw + skw_ref[nt] * xk
        else:
            xkw = xk
        t_ref[...] = xkw
        h_ref[...] = jnp.dot(xkw, wn_ref[...],
                             preferred_element_type=jnp.float32)

    return pl.pallas_call(
        body,
        grid=(G,),
        in_specs=[_p_spec(), _ea_spec(), _full_spec(DEP, D), _full_spec(D, D)]
                 + [_row_spec() for _ in range(nt)] + [_smem_spec(nt + 1)],
        out_specs=[_row_spec(), _row_spec()],
        out_shape=[jax.ShapeDtypeStruct((N, D), jnp.float32),
                   jax.ShapeDtypeStruct((N, D), jnp.float32)],
    )(p, eap, we, wn, *terms, skw)


def _tc_last(p, eap, we):
    def body(p_ref, ea_ref, we_ref, o_ref):
        ea = ea_ref[0] + ea_ref[1]
        agg = (p_ref[0] + p_ref[1]
               + jnp.dot(ea, we_ref[...], preferred_element_type=jnp.float32))
        o_ref[...] = jnp.maximum(agg, 0.0)

    return pl.pallas_call(
        body,
        grid=(G,),
        in_specs=[_p_spec(), _ea_spec(), _full_spec(DEP, D)],
        out_specs=_row_spec(),
        out_shape=jax.ShapeDtypeStruct((N, D), jnp.float32),
    )(p, eap, we)


def kernel(x, edge_index, edge_attr, params):
    L = params['layers']
    w = params['skip']

    npad = EP - E
    pad_pos = jnp.arange(npad, dtype=jnp.int32)
    pad_dst = N + pad_pos % (NP - N)
    pad_src = pad_pos % N
    src3 = jnp.concatenate(
        [edge_index[0], pad_src]).reshape(NW, NCHUNK, K)
    dst3 = jnp.concatenate(
        [edge_index[1], pad_dst]).reshape(NW, NCHUNK, K)
    eidx3 = jnp.concatenate(
        [jnp.arange(E, dtype=jnp.int32), pad_pos % E]
    ).reshape(NW, NCHUNK, K)
    ea2 = jnp.concatenate(
        [edge_attr,
         jnp.ones((E, 1), jnp.float32),
         jnp.zeros((E, DEP - DE - 1), jnp.float32)], axis=1)
    z128 = jnp.zeros((RPS, D), jnp.float32)

    def packed_we(l):
        p = L[l]
        return (jnp.zeros((DEP, D), jnp.float32)
                .at[:DE].set(p['We'])
                .at[DE].set(p['bn'] + p['be']))

    eap = _gather_segsum(ea2, eidx3, dst3, z128)

    agg_layers = [0, 1, 2, 3, 3, 4, 5]
    nxt_layers = [1, 2, 3, 3, 4, 5, 7]
    skips = [
        [],
        [w['w2_1'], w['w2_2']],
        [w['w3_1'], w['w3_2'], w['w3_3']],
        [w['w4_1'], w['w4_2'], w['w4_3'], w['w4_4']],
        [w['w5_1'], w['w5_2'], w['w5_3'], w['w5_4'], w['w5_5']],
        [w['w6_1'], w['w6_2'], w['w6_3'], w['w6_4'], w['w6_5'], w['w6_6']],
        [w['w7_1'], w['w7_2'], w['w7_3'], w['w7_4'], w['w7_5'], w['w7_6'],
         w['w7_7']],
    ]

    h = _tc_matmul(x, L[0]['Wn'])
    terms = []
    for k in range(7):
        p = _gather_segsum(h, src3, dst3, z128)
        skw = jnp.stack(skips[k]) if skips[k] else jnp.ones((1,), jnp.float32)
        xkw, h = _tc_step(p, eap, packed_we(agg_layers[k]),
                          L[nxt_layers[k]]['Wn'], terms, skw)
        terms.append(xkw)
    p = _gather_segsum(h, src3, dst3, z128)
    return _tc_last(p, eap, packed_we(7))


# speedup vs baseline: 3.6612x; 1.5633x over previous
import functools

import jax
import jax.numpy as jnp
from jax import lax
from jax.experimental import pallas as pl
from jax.experimental.pallas import tpu as pltpu
from jax.experimental.pallas import tpu_sc as plsc

N = 10000
E = 320000
D = 128
DE = 16
DEP = 128

NC = 2
NS = 16
NW = NC * NS
K = 80
EP = 327680
EPW = EP // NW
NCHUNK = EPW // K
NPH = 2
H = NCHUNK // NPH
NP = 10240
PAD_DST = N + 8
RPS = NP // NS

BR = 1000
G = N // BR


def _sc_mesh():
    return plsc.VectorSubcoreMesh(core_axis_name="c", subcore_axis_name="s")


def _gather_segsum(h, src3, dst3, zeros):
    @functools.partial(
        pl.kernel,
        mesh=_sc_mesh(),
        out_type=jax.ShapeDtypeStruct((NC, NP, D), jnp.float32),
        scratch_types=[
            pltpu.VMEM((H, K), jnp.int32),
            pltpu.VMEM((H, K), jnp.int32),
            pltpu.VMEM((K, D), jnp.float32),
            pltpu.VMEM((K, D), jnp.float32),
            pltpu.VMEM_SHARED((NP, D), jnp.float32),
            pltpu.SemaphoreType.DMA,
            pltpu.SemaphoreType.DMA,
        ],
    )
    def seg(h_hbm, src_hbm, dst_hbm, z_hbm, out_hbm,
            sidx, didx, rows0, rows1, acc, sem0, sem1):
        c = lax.axis_index("c")
        s = lax.axis_index("s")
        wid = s * NC + c
        pltpu.sync_copy(z_hbm, acc.at[pl.ds(s * RPS, RPS)])
        plsc.subcore_barrier()

        def phase(ph, carry):
            pltpu.sync_copy(src_hbm.at[wid, pl.ds(ph * H, H)], sidx)
            pltpu.sync_copy(dst_hbm.at[wid, pl.ds(ph * H, H)], didx)
            pltpu.async_copy(h_hbm.at[sidx.at[0]], rows0, sem0)
            pltpu.async_copy(h_hbm.at[sidx.at[1]], rows1, sem1)

            def body(i, carry2):
                j = 2 * i
                pltpu.make_async_copy(h_hbm.at[sidx.at[j]], rows0,
                                      sem0).wait()
                pltpu.sync_copy(rows0, acc.at[didx.at[j]], add=True)

                @pl.when(j + 2 < H)
                def _():
                    pltpu.async_copy(h_hbm.at[sidx.at[j + 2]], rows0, sem0)

                pltpu.make_async_copy(h_hbm.at[sidx.at[j + 1]], rows1,
                                      sem1).wait()
                pltpu.sync_copy(rows1, acc.at[didx.at[j + 1]], add=True)

                @pl.when(j + 3 < H)
                def _():
                    pltpu.async_copy(h_hbm.at[sidx.at[j + 3]], rows1, sem1)

                return carry2

            lax.fori_loop(0, H // 2, body, 0)
            return carry

        lax.fori_loop(0, NPH, phase, 0)
        plsc.subcore_barrier()
        pltpu.sync_copy(acc.at[pl.ds(s * RPS, RPS)],
                        out_hbm.at[c, pl.ds(s * RPS, RPS)])

    return seg(h, src3, dst3, zeros)


def _p_spec():
    return pl.BlockSpec((NC, BR, D), lambda i: (0, i, 0))


def _ea_spec():
    return pl.BlockSpec((NC, BR, DEP), lambda i: (0, i, 0))


def _row_spec(d=D):
    return pl.BlockSpec((BR, d), lambda i: (i, 0))


def _full_spec(a, b):
    return pl.BlockSpec((a, b), lambda i: (0, 0))


def _smem_spec(n):
    return pl.BlockSpec(memory_space=pltpu.SMEM)


def _tc_matmul(x, w):
    def body(x_ref, w_ref, o_ref):
        o_ref[...] = jnp.dot(x_ref[...], w_ref[...],
                             preferred_element_type=jnp.float32)

    return pl.pallas_call(
        body,
        grid=(G,),
        in_specs=[_row_spec(), _full_spec(D, D)],
        out_specs=_row_spec(),
        out_shape=jax.ShapeDtypeStruct((N, D), jnp.float32),
    )(x, w)


def _tc_step(p, eap, we, wn, terms, skw):
    nt = len(terms)

    def body(*refs):
        p_ref, ea_ref, we_ref, wn_ref = refs[:4]
        t_refs = refs[4:4 + nt]
        skw_ref = refs[4 + nt]
        t_ref, h_ref = refs[5 + nt:]
        ea = ea_ref[0] + ea_ref[1]
        agg = (p_ref[0] + p_ref[1]
               + jnp.dot(ea, we_ref[...], preferred_element_type=jnp.float32))
        xk = jnp.maximum(agg, 0.0)
        if nt:
            xkw = skw_ref[0] * t_refs[0][...]
            for j in range(1, nt):
                xkw = xkw + skw_ref[j] * t_refs[j][...]
            xkw = xkw + skw_ref[nt] * xk
        else:
            xkw = xk
        t_ref[...] = xkw
        h_ref[...] = jnp.dot(xkw, wn_ref[...],
                             preferred_element_type=jnp.float32)

    return pl.pallas_call(
        body,
        grid=(G,),
        in_specs=[_p_spec(), _ea_spec(), _full_spec(DEP, D), _full_spec(D, D)]
                 + [_row_spec() for _ in range(nt)] + [_smem_spec(nt + 1)],
        out_specs=[_row_spec(), _row_spec()],
        out_shape=[jax.ShapeDtypeStruct((N, D), jnp.float32),
                   jax.ShapeDtypeStruct((N, D), jnp.float32)],
    )(p, eap, we, wn, *terms, skw)


def _tc_last(p, eap, we):
    def body(p_ref, ea_ref, we_ref, o_ref):
        ea = ea_ref[0] + ea_ref[1]
        agg = (p_ref[0] + p_ref[1]
               + jnp.dot(ea, we_ref[...], preferred_element_type=jnp.float32))
        o_ref[...] = jnp.maximum(agg, 0.0)

    return pl.pallas_call(
        body,
        grid=(G,),
        in_specs=[_p_spec(), _ea_spec(), _full_spec(DEP, D)],
        out_specs=_row_spec(),
        out_shape=jax.ShapeDtypeStruct((N, D), jnp.float32),
    )(p, eap, we)


def kernel(x, edge_index, edge_attr, params):
    L = params['layers']
    w = params['skip']

    npad = EP - E
    pad_pos = jnp.arange(npad, dtype=jnp.int32)
    pad_dst = N + pad_pos % (NP - N)
    pad_src = pad_pos % N
    src3 = jnp.concatenate(
        [edge_index[0], pad_src]).reshape(NW, NCHUNK, K)
    dst3 = jnp.concatenate(
        [edge_index[1], pad_dst]).reshape(NW, NCHUNK, K)
    eidx3 = jnp.concatenate(
        [jnp.arange(E, dtype=jnp.int32), pad_pos % E]
    ).reshape(NW, NCHUNK, K)
    ea2 = jnp.concatenate(
        [edge_attr,
         jnp.ones((E, 1), jnp.float32),
         jnp.zeros((E, DEP - DE - 1), jnp.float32)], axis=1)
    z128 = jnp.zeros((RPS, D), jnp.float32)

    def packed_we(l):
        p = L[l]
        return (jnp.zeros((DEP, D), jnp.float32)
                .at[:DE].set(p['We'])
                .at[DE].set(p['bn'] + p['be']))

    eap = _gather_segsum(ea2, eidx3, dst3, z128)

    agg_layers = [0, 1, 2, 3, 3, 4, 5]
    nxt_layers = [1, 2, 3, 3, 4, 5, 7]
    skips = [
        [],
        [w['w2_1'], w['w2_2']],
        [w['w3_1'], w['w3_2'], w['w3_3']],
        [w['w4_1'], w['w4_2'], w['w4_3'], w['w4_4']],
        [w['w5_1'], w['w5_2'], w['w5_3'], w['w5_4'], w['w5_5']],
        [w['w6_1'], w['w6_2'], w['w6_3'], w['w6_4'], w['w6_5'], w['w6_6']],
        [w['w7_1'], w['w7_2'], w['w7_3'], w['w7_4'], w['w7_5'], w['w7_6'],
         w['w7_7']],
    ]

    h = _tc_matmul(x, L[0]['Wn'])
    terms = []
    for k in range(7):
        p = _gather_segsum(h, src3, dst3, z128)
        skw = jnp.stack(skips[k]) if skips[k] else jnp.ones((1,), jnp.float32)
        xkw, h = _tc_step(p, eap, packed_we(agg_layers[k]),
                          L[nxt_layers[k]]['Wn'], terms, skw)
        terms.append(xkw)
    p = _gather_segsum(h, src3, dst3, z128)
    return _tc_last(p, eap, packed_we(7))


# speedup vs baseline: 3.9396x; 1.0761x over previous
import functools

import jax
import jax.numpy as jnp
from jax import lax
from jax.experimental import pallas as pl
from jax.experimental.pallas import tpu as pltpu
from jax.experimental.pallas import tpu_sc as plsc

N = 10000
E = 320000
D = 128
DE = 16
DEP = 128

NC = 2
NS = 16
NW = NC * NS
K = 128
EP = 327680
EPW = EP // NW
NCHUNK = EPW // K
NPH = 2
H = NCHUNK // NPH
NP = 10240
PAD_DST = N + 8
RPS = NP // NS

BR = 1000
G = N // BR


def _sc_mesh():
    return plsc.VectorSubcoreMesh(core_axis_name="c", subcore_axis_name="s")


def _gather_segsum(h, src3, dst3, zeros):
    @functools.partial(
        pl.kernel,
        mesh=_sc_mesh(),
        out_type=jax.ShapeDtypeStruct((NC, NP, D), jnp.float32),
        scratch_types=[
            pltpu.VMEM((H, K), jnp.int32),
            pltpu.VMEM((H, K), jnp.int32),
            pltpu.VMEM((K, D), jnp.float32),
            pltpu.VMEM((K, D), jnp.float32),
            pltpu.VMEM_SHARED((NP, D), jnp.float32),
            pltpu.SemaphoreType.DMA,
            pltpu.SemaphoreType.DMA,
        ],
    )
    def seg(h_hbm, src_hbm, dst_hbm, z_hbm, out_hbm,
            sidx, didx, rows0, rows1, acc, sem0, sem1):
        c = lax.axis_index("c")
        s = lax.axis_index("s")
        wid = s * NC + c
        pltpu.sync_copy(z_hbm, acc.at[pl.ds(s * RPS, RPS)])
        plsc.subcore_barrier()

        def phase(ph, carry):
            pltpu.sync_copy(src_hbm.at[wid, pl.ds(ph * H, H)], sidx)
            pltpu.sync_copy(dst_hbm.at[wid, pl.ds(ph * H, H)], didx)
            pltpu.async_copy(h_hbm.at[sidx.at[0]], rows0, sem0)
            pltpu.async_copy(h_hbm.at[sidx.at[1]], rows1, sem1)

            def body(i, carry2):
                j = 2 * i
                pltpu.make_async_copy(h_hbm.at[sidx.at[j]], rows0,
                                      sem0).wait()
                pltpu.sync_copy(rows0, acc.at[didx.at[j]], add=True)

                @pl.when(j + 2 < H)
                def _():
                    pltpu.async_copy(h_hbm.at[sidx.at[j + 2]], rows0, sem0)

                pltpu.make_async_copy(h_hbm.at[sidx.at[j + 1]], rows1,
                                      sem1).wait()
                pltpu.sync_copy(rows1, acc.at[didx.at[j + 1]], add=True)

                @pl.when(j + 3 < H)
                def _():
                    pltpu.async_copy(h_hbm.at[sidx.at[j + 3]], rows1, sem1)

                return carry2

            lax.fori_loop(0, H // 2, body, 0)
            return carry

        lax.fori_loop(0, NPH, phase, 0)
        plsc.subcore_barrier()
        pltpu.sync_copy(acc.at[pl.ds(s * RPS, RPS)],
                        out_hbm.at[c, pl.ds(s * RPS, RPS)])

    return seg(h, src3, dst3, zeros)


def _p_spec():
    return pl.BlockSpec((NC, BR, D), lambda i: (0, i, 0))


def _ea_spec():
    return pl.BlockSpec((NC, BR, DEP), lambda i: (0, i, 0))


def _row_spec(d=D):
    return pl.BlockSpec((BR, d), lambda i: (i, 0))


def _full_spec(a, b):
    return pl.BlockSpec((a, b), lambda i: (0, 0))


def _smem_spec(n):
    return pl.BlockSpec(memory_space=pltpu.SMEM)


def _tc_matmul(x, w):
    def body(x_ref, w_ref, o_ref):
        o_ref[...] = jnp.dot(x_ref[...], w_ref[...],
                             preferred_element_type=jnp.float32)

    return pl.pallas_call(
        body,
        grid=(G,),
        in_specs=[_row_spec(), _full_spec(D, D)],
        out_specs=_row_spec(),
        out_shape=jax.ShapeDtypeStruct((N, D), jnp.float32),
    )(x, w)


def _tc_step(p, eap, we, wn, terms, skw):
    nt = len(terms)

    def body(*refs):
        p_ref, ea_ref, we_ref, wn_ref = refs[:4]
        t_refs = refs[4:4 + nt]
        skw_ref = refs[4 + nt]
        t_ref, h_ref = refs[5 + nt:]
        ea = ea_ref[0] + ea_ref[1]
        agg = (p_ref[0] + p_ref[1]
               + jnp.dot(ea, we_ref[...], preferred_element_type=jnp.float32))
        xk = jnp.maximum(agg, 0.0)
        if nt:
            xkw = skw_ref[0] * t_refs[0][...]
            for j in range(1, nt):
                xkw = xkw + skw_ref[j] * t_refs[j][...]
            xkw = xkw + skw_ref[nt] * xk
        else:
            xkw = xk
        t_ref[...] = xkw
        h_ref[...] = jnp.dot(xkw, wn_ref[...],
                             preferred_element_type=jnp.float32)

    return pl.pallas_call(
        body,
        grid=(G,),
        in_specs=[_p_spec(), _ea_spec(), _full_spec(DEP, D), _full_spec(D, D)]
                 + [_row_spec() for _ in range(nt)] + [_smem_spec(nt + 1)],
        out_specs=[_row_spec(), _row_spec()],
        out_shape=[jax.ShapeDtypeStruct((N, D), jnp.float32),
                   jax.ShapeDtypeStruct((N, D), jnp.float32)],
    )(p, eap, we, wn, *terms, skw)


def _tc_last(p, eap, we):
    def body(p_ref, ea_ref, we_ref, o_ref):
        ea = ea_ref[0] + ea_ref[1]
        agg = (p_ref[0] + p_ref[1]
               + jnp.dot(ea, we_ref[...], preferred_element_type=jnp.float32))
        o_ref[...] = jnp.maximum(agg, 0.0)

    return pl.pallas_call(
        body,
        grid=(G,),
        in_specs=[_p_spec(), _ea_spec(), _full_spec(DEP, D)],
        out_specs=_row_spec(),
        out_shape=jax.ShapeDtypeStruct((N, D), jnp.float32),
    )(p, eap, we)


def kernel(x, edge_index, edge_attr, params):
    L = params['layers']
    w = params['skip']

    npad = EP - E
    pad_pos = jnp.arange(npad, dtype=jnp.int32)
    pad_dst = N + pad_pos % (NP - N)
    pad_src = pad_pos % N
    src3 = jnp.concatenate(
        [edge_index[0], pad_src]).reshape(NW, NCHUNK, K)
    dst3 = jnp.concatenate(
        [edge_index[1], pad_dst]).reshape(NW, NCHUNK, K)
    eidx3 = jnp.concatenate(
        [jnp.arange(E, dtype=jnp.int32), pad_pos % E]
    ).reshape(NW, NCHUNK, K)
    ea2 = jnp.concatenate(
        [edge_attr,
         jnp.ones((E, 1), jnp.float32),
         jnp.zeros((E, DEP - DE - 1), jnp.float32)], axis=1)
    z128 = jnp.zeros((RPS, D), jnp.float32)

    def packed_we(l):
        p = L[l]
        return (jnp.zeros((DEP, D), jnp.float32)
                .at[:DE].set(p['We'])
                .at[DE].set(p['bn'] + p['be']))

    eap = _gather_segsum(ea2, eidx3, dst3, z128)

    agg_layers = [0, 1, 2, 3, 3, 4, 5]
    nxt_layers = [1, 2, 3, 3, 4, 5, 7]
    skips = [
        [],
        [w['w2_1'], w['w2_2']],
        [w['w3_1'], w['w3_2'], w['w3_3']],
        [w['w4_1'], w['w4_2'], w['w4_3'], w['w4_4']],
        [w['w5_1'], w['w5_2'], w['w5_3'], w['w5_4'], w['w5_5']],
        [w['w6_1'], w['w6_2'], w['w6_3'], w['w6_4'], w['w6_5'], w['w6_6']],
        [w['w7_1'], w['w7_2'], w['w7_3'], w['w7_4'], w['w7_5'], w['w7_6'],
         w['w7_7']],
    ]

    h = _tc_matmul(x, L[0]['Wn'])
    terms = []
    for k in range(7):
        p = _gather_segsum(h, src3, dst3, z128)
        skw = jnp.stack(skips[k]) if skips[k] else jnp.ones((1,), jnp.float32)
        xkw, h = _tc_step(p, eap, packed_we(agg_layers[k]),
                          L[nxt_layers[k]]['Wn'], terms, skw)
        terms.append(xkw)
    p = _gather_segsum(h, src3, dst3, z128)
    return _tc_last(p, eap, packed_we(7))


# speedup vs baseline: 4.0235x; 1.0213x over previous
import functools

import jax
import jax.numpy as jnp
from jax import lax
from jax.experimental import pallas as pl
from jax.experimental.pallas import tpu as pltpu
from jax.experimental.pallas import tpu_sc as plsc

N = 10000
E = 320000
D = 128
DE = 16
DEP = 128

NC = 2
NS = 16
NW = NC * NS
K = 128
EP = 327680
EPW = EP // NW
NCHUNK = EPW // K
NPH = 2
H = NCHUNK // NPH
NP = 10240
PAD_DST = N + 8
RPS = NP // NS

BR = 1000
G = N // BR


def _sc_mesh():
    return plsc.VectorSubcoreMesh(core_axis_name="c", subcore_axis_name="s")


def _gather_segsum(h, src3, dst3, zeros, linear=False):
    @functools.partial(
        pl.kernel,
        mesh=_sc_mesh(),
        out_type=jax.ShapeDtypeStruct((NC, NP, D), jnp.float32),
        scratch_types=[
            pltpu.VMEM((H, K), jnp.int32),
            pltpu.VMEM((H, K), jnp.int32),
            pltpu.VMEM((K, D), jnp.float32),
            pltpu.VMEM((K, D), jnp.float32),
            pltpu.VMEM_SHARED((NP, D), jnp.float32),
            pltpu.SemaphoreType.DMA,
            pltpu.SemaphoreType.DMA,
        ],
    )
    def seg(h_hbm, src_hbm, dst_hbm, z_hbm, out_hbm,
            sidx, didx, rows0, rows1, acc, sem0, sem1):
        c = lax.axis_index("c")
        s = lax.axis_index("s")
        wid = s * NC + c
        pltpu.sync_copy(z_hbm, acc.at[pl.ds(s * RPS, RPS)])
        plsc.subcore_barrier()

        def phase(ph, carry):
            if not linear:
                pltpu.sync_copy(src_hbm.at[wid, pl.ds(ph * H, H)], sidx)
            pltpu.sync_copy(dst_hbm.at[wid, pl.ds(ph * H, H)], didx)

            def gsrc(jj):
                if linear:
                    off = pl.multiple_of((wid * NCHUNK + ph * H + jj) * K, 8)
                    return h_hbm.at[pl.ds(off, K)]
                return h_hbm.at[sidx.at[jj]]

            pltpu.async_copy(gsrc(0), rows0, sem0)
            pltpu.async_copy(gsrc(1), rows1, sem1)

            def body(i, carry2):
                j = 2 * i
                pltpu.make_async_copy(gsrc(j), rows0, sem0).wait()
                pltpu.sync_copy(rows0, acc.at[didx.at[j]], add=True)

                @pl.when(j + 2 < H)
                def _():
                    pltpu.async_copy(gsrc(j + 2), rows0, sem0)

                pltpu.make_async_copy(gsrc(j + 1), rows1, sem1).wait()
                pltpu.sync_copy(rows1, acc.at[didx.at[j + 1]], add=True)

                @pl.when(j + 3 < H)
                def _():
                    pltpu.async_copy(gsrc(j + 3), rows1, sem1)

                return carry2

            lax.fori_loop(0, H // 2, body, 0)
            return carry

        lax.fori_loop(0, NPH, phase, 0)
        plsc.subcore_barrier()
        pltpu.sync_copy(acc.at[pl.ds(s * RPS, RPS)],
                        out_hbm.at[c, pl.ds(s * RPS, RPS)])

    return seg(h, src3, dst3, zeros)


def _p_spec():
    return pl.BlockSpec((NC, BR, D), lambda i: (0, i, 0))


def _ea_spec():
    return pl.BlockSpec((NC, BR, DEP), lambda i: (0, i, 0))


def _row_spec(d=D):
    return pl.BlockSpec((BR, d), lambda i: (i, 0))


def _full_spec(a, b):
    return pl.BlockSpec((a, b), lambda i: (0, 0))


def _smem_spec(n):
    return pl.BlockSpec(memory_space=pltpu.SMEM)


def _tc_matmul(x, w):
    def body(x_ref, w_ref, o_ref):
        o_ref[...] = jnp.dot(x_ref[...], w_ref[...],
                             preferred_element_type=jnp.float32)

    return pl.pallas_call(
        body,
        grid=(G,),
        in_specs=[_row_spec(), _full_spec(D, D)],
        out_specs=_row_spec(),
        out_shape=jax.ShapeDtypeStruct((N, D), jnp.float32),
    )(x, w)


def _tc_step(p, eap, we, wn, terms, skw):
    nt = len(terms)

    def body(*refs):
        p_ref, ea_ref, we_ref, wn_ref = refs[:4]
        t_refs = refs[4:4 + nt]
        skw_ref = refs[4 + nt]
        t_ref, h_ref = refs[5 + nt:]
        ea = ea_ref[0] + ea_ref[1]
        agg = (p_ref[0] + p_ref[1]
               + jnp.dot(ea, we_ref[...], preferred_element_type=jnp.float32))
        xk = jnp.maximum(agg, 0.0)
        if nt:
            xkw = skw_ref[0] * t_refs[0][...]
            for j in range(1, nt):
                xkw = xkw + skw_ref[j] * t_refs[j][...]
            xkw = xkw + skw_ref[nt] * xk
        else:
            xkw = xk
        t_ref[...] = xkw
        h_ref[...] = jnp.dot(xkw, wn_ref[...],
                             preferred_element_type=jnp.float32)

    return pl.pallas_call(
        body,
        grid=(G,),
        in_specs=[_p_spec(), _ea_spec(), _full_spec(DEP, D), _full_spec(D, D)]
                 + [_row_spec() for _ in range(nt)] + [_smem_spec(nt + 1)],
        out_specs=[_row_spec(), _row_spec()],
        out_shape=[jax.ShapeDtypeStruct((N, D), jnp.float32),
                   jax.ShapeDtypeStruct((N, D), jnp.float32)],
    )(p, eap, we, wn, *terms, skw)


def _tc_last(p, eap, we):
    def body(p_ref, ea_ref, we_ref, o_ref):
        ea = ea_ref[0] + ea_ref[1]
        agg = (p_ref[0] + p_ref[1]
               + jnp.dot(ea, we_ref[...], preferred_element_type=jnp.float32))
        o_ref[...] = jnp.maximum(agg, 0.0)

    return pl.pallas_call(
        body,
        grid=(G,),
        in_specs=[_p_spec(), _ea_spec(), _full_spec(DEP, D)],
        out_specs=_row_spec(),
        out_shape=jax.ShapeDtypeStruct((N, D), jnp.float32),
    )(p, eap, we)


def kernel(x, edge_index, edge_attr, params):
    L = params['layers']
    w = params['skip']

    npad = EP - E
    pad_pos = jnp.arange(npad, dtype=jnp.int32)
    pad_dst = N + pad_pos % (NP - N)
    pad_src = pad_pos % N
    src3 = jnp.concatenate(
        [edge_index[0], pad_src]).reshape(NW, NCHUNK, K)
    dst3 = jnp.concatenate(
        [edge_index[1], pad_dst]).reshape(NW, NCHUNK, K)
    ea2 = jnp.concatenate(
        [jnp.concatenate([edge_attr, jnp.zeros((npad, DE), jnp.float32)]),
         jnp.ones((EP, 1), jnp.float32),
         jnp.zeros((EP, DEP - DE - 1), jnp.float32)], axis=1)
    z128 = jnp.zeros((RPS, D), jnp.float32)

    def packed_we(l):
        p = L[l]
        return (jnp.zeros((DEP, D), jnp.float32)
                .at[:DE].set(p['We'])
                .at[DE].set(p['bn'] + p['be']))

    eap = _gather_segsum(ea2, src3, dst3, z128, linear=True)

    agg_layers = [0, 1, 2, 3, 3, 4, 5]
    nxt_layers = [1, 2, 3, 3, 4, 5, 7]
    skips = [
        [],
        [w['w2_1'], w['w2_2']],
        [w['w3_1'], w['w3_2'], w['w3_3']],
        [w['w4_1'], w['w4_2'], w['w4_3'], w['w4_4']],
        [w['w5_1'], w['w5_2'], w['w5_3'], w['w5_4'], w['w5_5']],
        [w['w6_1'], w['w6_2'], w['w6_3'], w['w6_4'], w['w6_5'], w['w6_6']],
        [w['w7_1'], w['w7_2'], w['w7_3'], w['w7_4'], w['w7_5'], w['w7_6'],
         w['w7_7']],
    ]

    h = _tc_matmul(x, L[0]['Wn'])
    terms = []
    for k in range(7):
        p = _gather_segsum(h, src3, dst3, z128)
        skw = jnp.stack(skips[k]) if skips[k] else jnp.ones((1,), jnp.float32)
        xkw, h = _tc_step(p, eap, packed_we(agg_layers[k]),
                          L[nxt_layers[k]]['Wn'], terms, skw)
        terms.append(xkw)
    p = _gather_segsum(h, src3, dst3, z128)
    return _tc_last(p, eap, packed_we(7))
